# baseline (speedup 1.0000x reference)
import functools

import numpy as np
import jax
import jax.numpy as jnp
from jax import lax
from jax.experimental import pallas as pl
from jax.experimental.pallas import tpu as pltpu
from jax.experimental.pallas import tpu_sc as plsc

H = W = 224
N = H * W
_RADII = [1, 2, 3, 4, 5, 6]
_RW = [1.0, 0.6, 0.4, 0.3, 0.2, 0.1]
_TAU = 0.15


def _build_static():
    d = {}
    for r, w in zip(_RADII, _RW):
        for dy in range(-r, r + 1):
            for dx in range(-r, r + 1):
                if (dx == 0 and dy == 0) or dx * dx + dy * dy > r * r:
                    continue
                d[(dy, dx)] = d.get((dy, dx), 0.0) + w
    offs = sorted(d.items(), key=lambda kv: kv[0][0] * W + kv[0][1])
    slots = []
    diag_j = None
    for (dy, dx), w in offs:
        if dy * W + dx > 0 and diag_j is None:
            diag_j = len(slots)
            slots.append((0, 0, None))
        slots.append((dy, dx, w))
    if diag_j is None:
        diag_j = len(slots)
        slots.append((0, 0, None))
    k = len(slots)

    dys = np.array([s[0] for s in slots], np.int64)
    dxs = np.array([s[1] for s in slots], np.int64)
    deltas = dys * W + dxs

    yy, xx = np.meshgrid(np.arange(H, dtype=np.int64),
                         np.arange(W, dtype=np.int64), indexing="ij")
    yf = yy.reshape(-1)
    xf = xx.reshape(-1)
    ny = yf[:, None] + dys[None, :]
    nx = xf[:, None] + dxs[None, :]
    valid = (ny >= 0) & (ny < H) & (nx >= 0) & (nx < W)

    pix = np.arange(N, dtype=np.int64)
    rows = np.broadcast_to(pix[:, None], (N, k))[valid]
    cols = (pix[:, None] + deltas[None, :])[valid]
    jj = np.broadcast_to(np.arange(k, dtype=np.int64)[None, :], (N, k))
    d2idx = (yf[:, None] * k + jj) * W + xf[:, None]
    sel = d2idx[valid]

    indices = np.stack([rows, cols]).astype(np.int32)
    return slots, diag_j, k, indices, sel.astype(np.int32)


_SLOTS, _DIAG_J, _K, _INDICES, _SEL = _build_static()
_NNZ = _SEL.shape[0]

_BY = 8
_WPAD = 256


def _stencil_body(planes_ref, out_ref):
    yb = pl.program_id(0) * _BY
    tall = [planes_ref[ch, pl.ds(yb, _BY + 16), :] for ch in range(3)]
    ctr = [t[6:6 + _BY, 6:6 + W] for t in tall]
    rowi = jax.lax.broadcasted_iota(jnp.int32, (_BY, W), 0) + yb
    coli = jax.lax.broadcasted_iota(jnp.int32, (_BY, W), 1)
    acc = jnp.zeros((_BY, W), jnp.float32)
    for j, (dy, dx, w) in enumerate(_SLOTS):
        if w is None:
            continue
        sh = [t[6 + dy:6 + dy + _BY, 6 + dx:6 + dx + W] for t in tall]
        d2 = ((ctr[0] - sh[0]) ** 2 + (ctr[1] - sh[1]) ** 2
              + (ctr[2] - sh[2]) ** 2)
        ew = w * jnp.exp(jnp.sqrt(d2) * (-1.0 / _TAU))
        ok = (coli >= -dx) & (coli < W - dx)
        if dy > 0:
            ok &= rowi < H - dy
        elif dy < 0:
            ok &= rowi >= -dy
        ewm = jnp.where(ok, ew, 0.0)
        acc = acc + ewm
        out_ref[:, j, :] = -ewm
    out_ref[:, _DIAG_J, :] = acc


def _dense_weights(image, interpret=False):
    img = image.astype(jnp.float32)
    planes = jnp.transpose(img, (2, 0, 1))
    planes = jnp.pad(planes, ((0, 0), (6, _BY + 10), (6, _WPAD - W - 6)))
    return pl.pallas_call(
        _stencil_body,
        grid=(H // _BY,),
        in_specs=[pl.BlockSpec((3, H + _BY + 16, _WPAD), lambda i: (0, 0, 0))],
        out_specs=pl.BlockSpec((_BY, _K, W), lambda i: (i, 0, 0)),
        out_shape=jax.ShapeDtypeStruct((H, _K, W), jnp.float32),
        interpret=interpret,
    )(planes)


_LANES = 128
_ROWS = _NNZ // _LANES
_NW = 32
_G = 40
_ROWS_PER_W = 1440
_ROWS_PAD = _ROWS_PER_W * _NW
_HALF = _ROWS_PER_W // 2
_PAIRS = _HALF // (2 * _G)

_SEL_PAD = np.zeros((_ROWS_PAD, _LANES), np.int32)
_SEL_PAD.reshape(-1)[:_NNZ] = _SEL


def _sc_compact(d2_flat, sel):
    mesh = plsc.VectorSubcoreMesh(core_axis_name="c", subcore_axis_name="s")

    @functools.partial(
        pl.kernel,
        mesh=mesh,
        out_type=jax.ShapeDtypeStruct((_ROWS_PAD, _LANES), jnp.float32),
        scratch_types=[
            pltpu.VMEM((_HALF, _LANES), jnp.int32),
            pltpu.VMEM((_G, _LANES), jnp.float32),
            pltpu.VMEM((_G, _LANES), jnp.float32),
            pltpu.SemaphoreType.DMA,
            pltpu.SemaphoreType.DMA,
            pltpu.SemaphoreType.DMA,
        ],
    )
    def gather_kernel(d2_hbm, sel_hbm, out_hbm, idx_v, da, db, gsem, oa, ob):
        wid = lax.axis_index("s") * 2 + lax.axis_index("c")
        base = wid * _ROWS_PER_W

        for half in range(2):
            hb = base + half * _HALF
            pltpu.sync_copy(sel_hbm.at[pl.ds(hb, _HALF)], idx_v)

            def pair(i, carry):
                r0 = i * 2 * _G
                cps = [pltpu.async_copy(d2_hbm.at[idx_v.at[r0 + r]],
                                        da.at[r], gsem) for r in range(_G)]
                for c in cps:
                    c.wait()
                st_a = pltpu.async_copy(da, out_hbm.at[pl.ds(hb + r0, _G)], oa)
                r1 = r0 + _G
                cps = [pltpu.async_copy(d2_hbm.at[idx_v.at[r1 + r]],
                                        db.at[r], gsem) for r in range(_G)]
                for c in cps:
                    c.wait()
                st_b = pltpu.async_copy(db, out_hbm.at[pl.ds(hb + r1, _G)], ob)
                st_a.wait()
                st_b.wait()
                return carry

            lax.fori_loop(0, _PAIRS, pair, 0)

    return gather_kernel(d2_flat, sel)


def kernel(image):
    d2 = _dense_weights(image)
    out = _sc_compact(d2.reshape(-1), jnp.asarray(_SEL_PAD))
    vals = out.reshape(-1)[:_NNZ]
    return jnp.asarray(_INDICES), vals


# speedup vs baseline: 3.9186x; 3.9186x over previous
import functools

import numpy as np
import jax
import jax.numpy as jnp
from jax import lax
from jax.experimental import pallas as pl
from jax.experimental.pallas import tpu as pltpu
from jax.experimental.pallas import tpu_sc as plsc

H = W = 224
N = H * W
_RADII = [1, 2, 3, 4, 5, 6]
_RW = [1.0, 0.6, 0.4, 0.3, 0.2, 0.1]
_TAU = 0.15


def _build_static():
    d = {}
    for r, w in zip(_RADII, _RW):
        for dy in range(-r, r + 1):
            for dx in range(-r, r + 1):
                if (dx == 0 and dy == 0) or dx * dx + dy * dy > r * r:
                    continue
                d[(dy, dx)] = d.get((dy, dx), 0.0) + w
    offs = sorted(d.items(), key=lambda kv: kv[0][0] * W + kv[0][1])
    slots = []
    diag_j = None
    for (dy, dx), w in offs:
        if dy * W + dx > 0 and diag_j is None:
            diag_j = len(slots)
            slots.append((0, 0, None))
        slots.append((dy, dx, w))
    if diag_j is None:
        diag_j = len(slots)
        slots.append((0, 0, None))
    k = len(slots)

    dys = np.array([s[0] for s in slots], np.int64)
    dxs = np.array([s[1] for s in slots], np.int64)
    deltas = dys * W + dxs

    yy, xx = np.meshgrid(np.arange(H, dtype=np.int64),
                         np.arange(W, dtype=np.int64), indexing="ij")
    yf = yy.reshape(-1)
    xf = xx.reshape(-1)
    ny = yf[:, None] + dys[None, :]
    nx = xf[:, None] + dxs[None, :]
    valid = (ny >= 0) & (ny < H) & (nx >= 0) & (nx < W)

    pix = np.arange(N, dtype=np.int64)
    rows = np.broadcast_to(pix[:, None], (N, k))[valid]
    cols = (pix[:, None] + deltas[None, :])[valid]
    jj = np.broadcast_to(np.arange(k, dtype=np.int64)[None, :], (N, k))
    d2idx = (yf[:, None] * k + jj) * W + xf[:, None]
    sel = d2idx[valid]

    indices = np.stack([rows, cols]).astype(np.int32)
    rowlens = valid.reshape(H, W * k).sum(axis=1).astype(np.int64)
    return slots, diag_j, k, indices, sel.astype(np.int32), rowlens


_SLOTS, _DIAG_J, _K, _INDICES, _SEL, _ROWLENS = _build_static()
_NNZ = _SEL.shape[0]

_BY = 8
_WPAD = 256


def _stencil_body(planes_ref, out_ref):
    yb = pl.program_id(0) * _BY
    tall = [planes_ref[ch, pl.ds(yb, _BY + 16), :] for ch in range(3)]
    ctr = [t[6:6 + _BY, 6:6 + W] for t in tall]
    rowi = jax.lax.broadcasted_iota(jnp.int32, (_BY, W), 0) + yb
    coli = jax.lax.broadcasted_iota(jnp.int32, (_BY, W), 1)
    acc = jnp.zeros((_BY, W), jnp.float32)
    for j, (dy, dx, w) in enumerate(_SLOTS):
        if w is None:
            continue
        sh = [t[6 + dy:6 + dy + _BY, 6 + dx:6 + dx + W] for t in tall]
        d2 = ((ctr[0] - sh[0]) ** 2 + (ctr[1] - sh[1]) ** 2
              + (ctr[2] - sh[2]) ** 2)
        ew = w * jnp.exp(jnp.sqrt(d2) * (-1.0 / _TAU))
        ok = (coli >= -dx) & (coli < W - dx)
        if dy > 0:
            ok &= rowi < H - dy
        elif dy < 0:
            ok &= rowi >= -dy
        ewm = jnp.where(ok, ew, 0.0)
        acc = acc + ewm
        out_ref[:, j, :] = -ewm
    out_ref[:, _DIAG_J, :] = acc


def _dense_weights(image, interpret=False):
    img = image.astype(jnp.float32)
    planes = jnp.transpose(img, (2, 0, 1))
    planes = jnp.pad(planes, ((0, 0), (6, _BY + 10), (6, _WPAD - W - 6)))
    return pl.pallas_call(
        _stencil_body,
        grid=(H // _BY,),
        in_specs=[pl.BlockSpec((3, H + _BY + 16, _WPAD), lambda i: (0, 0, 0))],
        out_specs=pl.BlockSpec((_BY, _K, W), lambda i: (i, 0, 0)),
        out_shape=jax.ShapeDtypeStruct((H, _K, W), jnp.float32),
        interpret=interpret,
    )(planes)


_LANES = 128
_NW = 32
_PLANE = _K * W
_RING = 2 * _PLANE
_SB = 80
_SUBBLK = _SB * _LANES
_NSB = 18
_ROWS_PER_W = _SB * _NSB
_ROWS_PAD = _ROWS_PER_W * _NW
_NPAD = _ROWS_PAD * _LANES

_SELM = np.zeros(_NPAD, np.int32)
_SELM[:_NNZ] = _SEL % _RING

_T = np.concatenate([[0], np.cumsum(_ROWLENS)]).astype(np.int64)
_C0 = int(_T[6])
_LMID = int(_T[7] - _T[6])
_C218 = int(_T[218])
assert all(int(_T[y + 1] - _T[y]) == _LMID for y in range(6, 218))
assert int(_ROWLENS.min()) > _SUBBLK
_TTOP = [int(_T[y]) for y in range(1, 6)]
_TBOT = [int(_T[y]) for y in range(219, 225)]


def _yfun(o):
    y_top = sum((o >= t).astype(jnp.int32) for t in _TTOP)
    y_mid = (6 + (o - _C0) // _LMID).astype(jnp.int32)
    y_bot = 218 + sum((o >= t).astype(jnp.int32) for t in _TBOT)
    return jnp.where(o < _C0, y_top, jnp.where(o < _C218, y_mid, y_bot))


def _sc_compact(d2pad, selm):
    mesh = plsc.VectorSubcoreMesh(core_axis_name="c", subcore_axis_name="s")

    @functools.partial(
        pl.kernel,
        mesh=mesh,
        out_type=jax.ShapeDtypeStruct((_NPAD,), jnp.float32),
        compiler_params=pltpu.CompilerParams(needs_layout_passes=False),
        scratch_types=[
            pltpu.VMEM((_RING,), jnp.float32),
            pltpu.VMEM((_SUBBLK,), jnp.int32),
            pltpu.VMEM((_SUBBLK,), jnp.float32),
        ],
    )
    def gather_kernel(d2_hbm, selm_hbm, out_hbm, plane_v, idx_v, dat_v):
        wid = lax.axis_index("s") * 2 + lax.axis_index("c")
        obase = wid * (_ROWS_PER_W * _LANES)
        y0 = _yfun(obase)
        pltpu.sync_copy(d2_hbm.at[pl.ds(y0 * _PLANE, _PLANE)],
                        plane_v.at[pl.ds((y0 % 2) * _PLANE, _PLANE)])

        def sub_block(i, ycur):
            o = obase + i * _SUBBLK
            pltpu.sync_copy(selm_hbm.at[pl.ds(o, _SUBBLK)], idx_v)
            ynew = _yfun(o + _SUBBLK - 1)

            @pl.when(ynew != ycur)
            def _():
                pltpu.sync_copy(d2_hbm.at[pl.ds(ynew * _PLANE, _PLANE)],
                                plane_v.at[pl.ds((ynew % 2) * _PLANE, _PLANE)])

            def gv(v, c):
                for u in range(8):
                    off = v * 128 + u * 16
                    idx = idx_v[pl.ds(off, 16)]
                    dat_v[pl.ds(off, 16)] = plsc.load_gather(plane_v, [idx])
                return c

            lax.fori_loop(0, _SUBBLK // 128, gv, 0)
            pltpu.sync_copy(dat_v, out_hbm.at[pl.ds(o, _SUBBLK)])
            return ynew

        lax.fori_loop(0, _NSB, sub_block, y0)

    return gather_kernel(d2pad, selm)


def kernel(image):
    d2 = _dense_weights(image)
    d2pad = jnp.concatenate([d2.reshape(-1),
                             jnp.zeros((_PLANE,), jnp.float32)])
    out = _sc_compact(d2pad, jnp.asarray(_SELM))
    vals = out[:_NNZ]
    return jnp.asarray(_INDICES), vals


# speedup vs baseline: 4.1685x; 1.0638x over previous
import functools

import numpy as np
import jax
import jax.numpy as jnp
from jax import lax
from jax.experimental import pallas as pl
from jax.experimental.pallas import tpu as pltpu
from jax.experimental.pallas import tpu_sc as plsc

H = W = 224
N = H * W
_RADII = [1, 2, 3, 4, 5, 6]
_RW = [1.0, 0.6, 0.4, 0.3, 0.2, 0.1]
_TAU = 0.15


def _build_static():
    d = {}
    for r, w in zip(_RADII, _RW):
        for dy in range(-r, r + 1):
            for dx in range(-r, r + 1):
                if (dx == 0 and dy == 0) or dx * dx + dy * dy > r * r:
                    continue
                d[(dy, dx)] = d.get((dy, dx), 0.0) + w
    offs = sorted(d.items(), key=lambda kv: kv[0][0] * W + kv[0][1])
    slots = []
    diag_j = None
    for (dy, dx), w in offs:
        if dy * W + dx > 0 and diag_j is None:
            diag_j = len(slots)
            slots.append((0, 0, None))
        slots.append((dy, dx, w))
    if diag_j is None:
        diag_j = len(slots)
        slots.append((0, 0, None))
    k = len(slots)

    dys = np.array([s[0] for s in slots], np.int64)
    dxs = np.array([s[1] for s in slots], np.int64)
    deltas = dys * W + dxs

    yy, xx = np.meshgrid(np.arange(H, dtype=np.int64),
                         np.arange(W, dtype=np.int64), indexing="ij")
    yf = yy.reshape(-1)
    xf = xx.reshape(-1)
    ny = yf[:, None] + dys[None, :]
    nx = xf[:, None] + dxs[None, :]
    valid = (ny >= 0) & (ny < H) & (nx >= 0) & (nx < W)

    pix = np.arange(N, dtype=np.int64)
    rows = np.broadcast_to(pix[:, None], (N, k))[valid]
    cols = (pix[:, None] + deltas[None, :])[valid]
    jj = np.broadcast_to(np.arange(k, dtype=np.int64)[None, :], (N, k))
    d2idx = (yf[:, None] * k + jj) * W + xf[:, None]
    sel = d2idx[valid]

    indices = np.stack([rows, cols]).astype(np.int32)
    rowlens = valid.reshape(H, W * k).sum(axis=1).astype(np.int64)
    return slots, diag_j, k, indices, sel.astype(np.int32), rowlens


_SLOTS, _DIAG_J, _K, _INDICES, _SEL, _ROWLENS = _build_static()
_NNZ = _SEL.shape[0]

_BY = 8
_WPAD = 256


def _stencil_body(planes_ref, out_ref):
    yb = pl.program_id(0) * _BY
    tall = [planes_ref[ch, pl.ds(yb, _BY + 16), :] for ch in range(3)]
    ctr = [t[6:6 + _BY, 6:6 + W] for t in tall]
    rowi = jax.lax.broadcasted_iota(jnp.int32, (_BY, W), 0) + yb
    coli = jax.lax.broadcasted_iota(jnp.int32, (_BY, W), 1)
    acc = jnp.zeros((_BY, W), jnp.float32)
    for j, (dy, dx, w) in enumerate(_SLOTS):
        if w is None:
            continue
        sh = [t[6 + dy:6 + dy + _BY, 6 + dx:6 + dx + W] for t in tall]
        d2 = ((ctr[0] - sh[0]) ** 2 + (ctr[1] - sh[1]) ** 2
              + (ctr[2] - sh[2]) ** 2)
        ew = w * jnp.exp(jnp.sqrt(d2) * (-1.0 / _TAU))
        ok = (coli >= -dx) & (coli < W - dx)
        if dy > 0:
            ok &= rowi < H - dy
        elif dy < 0:
            ok &= rowi >= -dy
        ewm = jnp.where(ok, ew, 0.0)
        acc = acc + ewm
        out_ref[:, j, :] = -ewm
    out_ref[:, _DIAG_J, :] = acc


def _dense_weights(image, interpret=False):
    img = image.astype(jnp.float32)
    planes = jnp.transpose(img, (2, 0, 1))
    planes = jnp.pad(planes, ((0, 0), (6, _BY + 10), (6, _WPAD - W - 6)))
    return pl.pallas_call(
        _stencil_body,
        grid=(H // _BY + 1,),
        in_specs=[pl.BlockSpec((3, H + _BY + 16, _WPAD), lambda i: (0, 0, 0))],
        out_specs=pl.BlockSpec((_BY, _K, W), lambda i: (i, 0, 0)),
        out_shape=jax.ShapeDtypeStruct((H + _BY, _K, W), jnp.float32),
        interpret=interpret,
    )(planes)


_LANES = 128
_NW = 32
_PLANE = _K * W
_RING = 2 * _PLANE
_SB = 80
_SUBBLK = _SB * _LANES
_NSB = 17
_ROWS_PER_W = _SB * _NSB
_OCLAMP = _NNZ - _SUBBLK

_SELM = (_SEL % _RING).astype(np.int32)

_T = np.concatenate([[0], np.cumsum(_ROWLENS)]).astype(np.int64)
_C0 = int(_T[6])
_LMID = int(_T[7] - _T[6])
_C218 = int(_T[218])
assert all(int(_T[y + 1] - _T[y]) == _LMID for y in range(6, 218))
assert int(_ROWLENS.min()) > _SUBBLK
_TTOP = [int(_T[y]) for y in range(1, 6)]
_TBOT = [int(_T[y]) for y in range(219, 225)]


def _yfun(o):
    y_top = sum((o >= t).astype(jnp.int32) for t in _TTOP)
    y_mid = (6 + (o - _C0) // _LMID).astype(jnp.int32)
    y_bot = 218 + sum((o >= t).astype(jnp.int32) for t in _TBOT)
    return jnp.where(o < _C0, y_top, jnp.where(o < _C218, y_mid, y_bot))


def _sc_compact(d2pad, selm):
    mesh = plsc.VectorSubcoreMesh(core_axis_name="c", subcore_axis_name="s")

    @functools.partial(
        pl.kernel,
        mesh=mesh,
        out_type=jax.ShapeDtypeStruct((_NNZ,), jnp.float32),
        compiler_params=pltpu.CompilerParams(needs_layout_passes=False),
        scratch_types=[
            pltpu.VMEM((_RING,), jnp.float32),
            pltpu.VMEM((_SUBBLK,), jnp.int32),
            pltpu.VMEM((_SUBBLK,), jnp.float32),
        ],
    )
    def gather_kernel(d2_hbm, selm_hbm, out_hbm, plane_v, idx_v, dat_v):
        wid = lax.axis_index("s") * 2 + lax.axis_index("c")
        obase = wid * (_ROWS_PER_W * _LANES)
        y0 = _yfun(obase)
        pltpu.sync_copy(d2_hbm.at[pl.ds(y0 * _PLANE, _PLANE)],
                        plane_v.at[pl.ds((y0 % 2) * _PLANE, _PLANE)])

        def sub_block(i, ycur):
            o = jnp.minimum(obase + i * _SUBBLK, _OCLAMP)
            pltpu.sync_copy(selm_hbm.at[pl.ds(o, _SUBBLK)], idx_v)
            ynew = _yfun(o + _SUBBLK - 1)

            @pl.when(ynew != ycur)
            def _():
                pltpu.sync_copy(d2_hbm.at[pl.ds(ynew * _PLANE, _PLANE)],
                                plane_v.at[pl.ds((ynew % 2) * _PLANE, _PLANE)])

            def gv(v, c):
                for u in range(8):
                    off = v * 128 + u * 16
                    idx = idx_v[pl.ds(off, 16)]
                    dat_v[pl.ds(off, 16)] = plsc.load_gather(plane_v, [idx])
                return c

            lax.fori_loop(0, _SUBBLK // 128, gv, 0)
            pltpu.sync_copy(dat_v, out_hbm.at[pl.ds(o, _SUBBLK)])
            return ynew

        lax.fori_loop(0, _NSB, sub_block, y0)

    return gather_kernel(d2pad, selm)


def kernel(image):
    d2pad = _dense_weights(image).reshape(-1)
    vals = _sc_compact(d2pad, jnp.asarray(_SELM))
    return jnp.asarray(_INDICES), vals


# speedup vs baseline: 6.1923x; 1.4855x over previous
import functools

import numpy as np
import jax
import jax.numpy as jnp
from jax import lax
from jax.experimental import pallas as pl
from jax.experimental.pallas import tpu as pltpu
from jax.experimental.pallas import tpu_sc as plsc

H = W = 224
N = H * W
_RADII = [1, 2, 3, 4, 5, 6]
_RW = [1.0, 0.6, 0.4, 0.3, 0.2, 0.1]
_TAU = 0.15


def _build_static():
    d = {}
    for r, w in zip(_RADII, _RW):
        for dy in range(-r, r + 1):
            for dx in range(-r, r + 1):
                if (dx == 0 and dy == 0) or dx * dx + dy * dy > r * r:
                    continue
                d[(dy, dx)] = d.get((dy, dx), 0.0) + w
    offs = sorted(d.items(), key=lambda kv: kv[0][0] * W + kv[0][1])
    slots = []
    diag_j = None
    for (dy, dx), w in offs:
        if dy * W + dx > 0 and diag_j is None:
            diag_j = len(slots)
            slots.append((0, 0, None))
        slots.append((dy, dx, w))
    if diag_j is None:
        diag_j = len(slots)
        slots.append((0, 0, None))
    k = len(slots)

    dys = np.array([s[0] for s in slots], np.int64)
    dxs = np.array([s[1] for s in slots], np.int64)
    deltas = dys * W + dxs

    yy, xx = np.meshgrid(np.arange(H, dtype=np.int64),
                         np.arange(W, dtype=np.int64), indexing="ij")
    yf = yy.reshape(-1)
    xf = xx.reshape(-1)
    ny = yf[:, None] + dys[None, :]
    nx = xf[:, None] + dxs[None, :]
    valid = (ny >= 0) & (ny < H) & (nx >= 0) & (nx < W)

    pix = np.arange(N, dtype=np.int64)
    rows = np.broadcast_to(pix[:, None], (N, k))[valid]
    cols = (pix[:, None] + deltas[None, :])[valid]
    jj = np.broadcast_to(np.arange(k, dtype=np.int64)[None, :], (N, k))
    d2idx = (yf[:, None] * k + jj) * 256 + xf[:, None]
    sel = d2idx[valid]

    indices = np.stack([rows, cols]).astype(np.int32)
    rowlens = valid.reshape(H, W * k).sum(axis=1).astype(np.int64)
    return slots, diag_j, k, indices, sel.astype(np.int32), rowlens


_SLOTS, _DIAG_J, _K, _INDICES, _SEL, _ROWLENS = _build_static()
_NNZ = _SEL.shape[0]

_BY = 32
_WPAD = 256
_WOUT = 256

_BY_GROUPS = {}
for _j, (_dy, _dx, _w) in enumerate(_SLOTS):
    if _w is not None:
        _BY_GROUPS.setdefault(_dy, []).append((_j, _dx, _w))


def _stencil_body(planes_ref, out_ref):
    yb = pl.program_id(0) * _BY
    tall = [planes_ref[ch, pl.ds(yb, _BY + 16), :] for ch in range(3)]
    ctr = [t[6:6 + _BY, 6:6 + W] for t in tall]
    rowi = jax.lax.broadcasted_iota(jnp.int32, (_BY, W), 0) + yb
    coli = jax.lax.broadcasted_iota(jnp.int32, (_BY, W), 1)
    one = jnp.ones((_BY, W), jnp.float32)
    zero = jnp.zeros((_BY, W), jnp.float32)
    oky = {dy: (jnp.where((rowi + dy >= 0) & (rowi + dy < H), one, zero)
                if dy else one)
           for dy in range(-6, 7)}
    okx = {dx: (jnp.where((coli + dx >= 0) & (coli + dx < W), one, zero)
                if dx else one)
           for dx in range(-6, 7)}
    accs = [zero, zero, zero, zero]
    for dy, group in sorted(_BY_GROUPS.items()):
        rsh = [t[6 + dy:6 + dy + _BY, :] for t in tall]
        for j, dx, w in group:
            sh = [r[:, 6 + dx:6 + dx + W] for r in rsh]
            d2 = ((ctr[0] - sh[0]) ** 2 + (ctr[1] - sh[1]) ** 2
                  + (ctr[2] - sh[2]) ** 2)
            ew = w * jnp.exp(jnp.sqrt(d2) * (-1.0 / _TAU))
            ewm = ew * (oky[dy] * okx[dx])
            accs[j % 4] = accs[j % 4] + ewm
            out_ref[:, j * _WOUT:j * _WOUT + W] = -ewm
    out_ref[:, _DIAG_J * _WOUT:_DIAG_J * _WOUT + W] = (
        (accs[0] + accs[1]) + (accs[2] + accs[3]))


def _dense_weights(image, interpret=False):
    img = image.astype(jnp.float32)
    planes = jnp.transpose(img, (2, 0, 1))
    planes = jnp.pad(planes, ((0, 0), (6, _BY + 10), (6, _WPAD - W - 6)))
    return pl.pallas_call(
        _stencil_body,
        grid=(H // _BY + 1,),
        in_specs=[pl.BlockSpec((3, H + _BY + 16, _WPAD), lambda i: (0, 0, 0))],
        out_specs=pl.BlockSpec((_BY, _K * _WOUT), lambda i: (i, 0)),
        out_shape=jax.ShapeDtypeStruct((H + _BY, _K * _WOUT), jnp.float32),
        interpret=interpret,
    )(planes)


_LANES = 128
_NW = 32
_PLANE = _K * _WOUT
_RING = 2 * _PLANE
_SB = 80
_SUBBLK = _SB * _LANES
_NSB = 17
_ROWS_PER_W = _SB * _NSB
_OCLAMP = _NNZ - _SUBBLK

_SELM = (_SEL % _RING).astype(np.int32)

_T = np.concatenate([[0], np.cumsum(_ROWLENS)]).astype(np.int64)
_C0 = int(_T[6])
_LMID = int(_T[7] - _T[6])
_C218 = int(_T[218])
assert all(int(_T[y + 1] - _T[y]) == _LMID for y in range(6, 218))
assert int(_ROWLENS.min()) > _SUBBLK
_TTOP = [int(_T[y]) for y in range(1, 6)]
_TBOT = [int(_T[y]) for y in range(219, 225)]


def _yfun(o):
    y_top = sum((o >= t).astype(jnp.int32) for t in _TTOP)
    y_mid = (6 + (o - _C0) // _LMID).astype(jnp.int32)
    y_bot = 218 + sum((o >= t).astype(jnp.int32) for t in _TBOT)
    return jnp.where(o < _C0, y_top, jnp.where(o < _C218, y_mid, y_bot))


def _sc_compact(d2pad, selm):
    mesh = plsc.VectorSubcoreMesh(core_axis_name="c", subcore_axis_name="s")

    @functools.partial(
        pl.kernel,
        mesh=mesh,
        out_type=jax.ShapeDtypeStruct((_NNZ,), jnp.float32),
        compiler_params=pltpu.CompilerParams(needs_layout_passes=False),
        scratch_types=[
            pltpu.VMEM((_RING,), jnp.float32),
            pltpu.VMEM((_SUBBLK,), jnp.int32),
            pltpu.VMEM((_SUBBLK,), jnp.float32),
        ],
    )
    def gather_kernel(d2_hbm, selm_hbm, out_hbm, plane_v, idx_v, dat_v):
        wid = lax.axis_index("s") * 2 + lax.axis_index("c")
        obase = wid * (_ROWS_PER_W * _LANES)
        y0 = _yfun(obase)
        pltpu.sync_copy(d2_hbm.at[pl.ds(y0 * _PLANE, _PLANE)],
                        plane_v.at[pl.ds((y0 % 2) * _PLANE, _PLANE)])

        def sub_block(i, ycur):
            o = jnp.minimum(obase + i * _SUBBLK, _OCLAMP)
            pltpu.sync_copy(selm_hbm.at[pl.ds(o, _SUBBLK)], idx_v)
            ynew = _yfun(o + _SUBBLK - 1)

            @pl.when(ynew != ycur)
            def _():
                pltpu.sync_copy(d2_hbm.at[pl.ds(ynew * _PLANE, _PLANE)],
                                plane_v.at[pl.ds((ynew % 2) * _PLANE, _PLANE)])

            def gv(v, c):
                for u in range(8):
                    off = v * 128 + u * 16
                    idx = idx_v[pl.ds(off, 16)]
                    dat_v[pl.ds(off, 16)] = plsc.load_gather(plane_v, [idx])
                return c

            lax.fori_loop(0, _SUBBLK // 128, gv, 0)
            pltpu.sync_copy(dat_v, out_hbm.at[pl.ds(o, _SUBBLK)])
            return ynew

        lax.fori_loop(0, _NSB, sub_block, y0)

    return gather_kernel(d2pad, selm)


def kernel(image):
    d2pad = _dense_weights(image).reshape(-1)
    vals = _sc_compact(d2pad, jnp.asarray(_SELM))
    return jnp.asarray(_INDICES), vals


# speedup vs baseline: 6.3511x; 1.0256x over previous
import functools

import numpy as np
import jax
import jax.numpy as jnp
from jax import lax
from jax.experimental import pallas as pl
from jax.experimental.pallas import tpu as pltpu
from jax.experimental.pallas import tpu_sc as plsc

H = W = 224
N = H * W
_RADII = [1, 2, 3, 4, 5, 6]
_RW = [1.0, 0.6, 0.4, 0.3, 0.2, 0.1]
_TAU = 0.15


def _build_static():
    d = {}
    for r, w in zip(_RADII, _RW):
        for dy in range(-r, r + 1):
            for dx in range(-r, r + 1):
                if (dx == 0 and dy == 0) or dx * dx + dy * dy > r * r:
                    continue
                d[(dy, dx)] = d.get((dy, dx), 0.0) + w
    offs = sorted(d.items(), key=lambda kv: kv[0][0] * W + kv[0][1])
    slots = []
    diag_j = None
    for (dy, dx), w in offs:
        if dy * W + dx > 0 and diag_j is None:
            diag_j = len(slots)
            slots.append((0, 0, None))
        slots.append((dy, dx, w))
    if diag_j is None:
        diag_j = len(slots)
        slots.append((0, 0, None))
    k = len(slots)

    dys = np.array([s[0] for s in slots], np.int64)
    dxs = np.array([s[1] for s in slots], np.int64)
    deltas = dys * W + dxs

    yy, xx = np.meshgrid(np.arange(H, dtype=np.int64),
                         np.arange(W, dtype=np.int64), indexing="ij")
    yf = yy.reshape(-1)
    xf = xx.reshape(-1)
    ny = yf[:, None] + dys[None, :]
    nx = xf[:, None] + dxs[None, :]
    valid = (ny >= 0) & (ny < H) & (nx >= 0) & (nx < W)

    pix = np.arange(N, dtype=np.int64)
    rows = np.broadcast_to(pix[:, None], (N, k))[valid]
    cols = (pix[:, None] + deltas[None, :])[valid]
    jj = np.broadcast_to(np.arange(k, dtype=np.int64)[None, :], (N, k))
    d2idx = (yf[:, None] * k + jj) * 256 + xf[:, None]
    sel = d2idx[valid]

    indices = np.stack([rows, cols]).astype(np.int32)
    rowlens = valid.reshape(H, W * k).sum(axis=1).astype(np.int64)
    return slots, diag_j, k, indices, sel.astype(np.int32), rowlens


_SLOTS, _DIAG_J, _K, _INDICES, _SEL, _ROWLENS = _build_static()
_NNZ = _SEL.shape[0]

_BY = 32
_WPAD = 256
_WOUT = 256

_BY_GROUPS = {}
for _j, (_dy, _dx, _w) in enumerate(_SLOTS):
    if _w is not None:
        _BY_GROUPS.setdefault(_dy, []).append((_j, _dx, _w))


def _stencil_body(planes_ref, out_ref):
    yb = pl.program_id(0) * _BY
    tall = [planes_ref[ch, pl.ds(yb, _BY + 16), :] for ch in range(3)]
    ctr = [t[6:6 + _BY, 6:6 + W] for t in tall]
    rowi = jax.lax.broadcasted_iota(jnp.int32, (_BY, W), 0) + yb
    coli = jax.lax.broadcasted_iota(jnp.int32, (_BY, W), 1)
    one = jnp.ones((_BY, W), jnp.float32)
    zero = jnp.zeros((_BY, W), jnp.float32)
    oky = {dy: (jnp.where((rowi + dy >= 0) & (rowi + dy < H), one, zero)
                if dy else one)
           for dy in range(-6, 7)}
    okx = {dx: (jnp.where((coli + dx >= 0) & (coli + dx < W), one, zero)
                if dx else one)
           for dx in range(-6, 7)}
    accs = [zero, zero, zero, zero]
    for dy, group in sorted(_BY_GROUPS.items()):
        rsh = [t[6 + dy:6 + dy + _BY, :] for t in tall]
        for j, dx, w in group:
            sh = [r[:, 6 + dx:6 + dx + W] for r in rsh]
            d2 = ((ctr[0] - sh[0]) ** 2 + (ctr[1] - sh[1]) ** 2
                  + (ctr[2] - sh[2]) ** 2)
            ew = w * jnp.exp(jnp.sqrt(d2) * (-1.0 / _TAU))
            ewm = ew * (oky[dy] * okx[dx])
            accs[j % 4] = accs[j % 4] + ewm
            out_ref[:, j * _WOUT:j * _WOUT + W] = -ewm
    out_ref[:, _DIAG_J * _WOUT:_DIAG_J * _WOUT + W] = (
        (accs[0] + accs[1]) + (accs[2] + accs[3]))


def _dense_weights(image, interpret=False):
    img = image.astype(jnp.float32)
    planes = jnp.transpose(img, (2, 0, 1))
    planes = jnp.pad(planes, ((0, 0), (6, _BY + 10), (6, _WPAD - W - 6)))
    return pl.pallas_call(
        _stencil_body,
        grid=(H // _BY + 1,),
        in_specs=[pl.BlockSpec((3, H + _BY + 16, _WPAD), lambda i: (0, 0, 0))],
        out_specs=pl.BlockSpec((_BY, _K * _WOUT), lambda i: (i, 0)),
        out_shape=jax.ShapeDtypeStruct((H + _BY, _K * _WOUT), jnp.float32),
        interpret=interpret,
    )(planes)


_LANES = 128
_NW = 32
_PLANE = _K * _WOUT
_RING = 2 * _PLANE
_SB = 80
_SUBBLK = _SB * _LANES
_NSB = 18
_ROWS_PER_W = 1360
_OCLAMP = _NNZ - _SUBBLK

_SELM = (_SEL % _RING).astype(np.int32)

_T = np.concatenate([[0], np.cumsum(_ROWLENS)]).astype(np.int64)
_C0 = int(_T[6])
_LMID = int(_T[7] - _T[6])
_C218 = int(_T[218])
assert all(int(_T[y + 1] - _T[y]) == _LMID for y in range(6, 218))
assert int(_ROWLENS.min()) > _SUBBLK
_TTOP = [int(_T[y]) for y in range(1, 6)]
_TBOT = [int(_T[y]) for y in range(219, 225)]


def _yfun(o):
    y_top = sum((o >= t).astype(jnp.int32) for t in _TTOP)
    y_mid = (6 + (o - _C0) // _LMID).astype(jnp.int32)
    y_bot = 218 + sum((o >= t).astype(jnp.int32) for t in _TBOT)
    return jnp.where(o < _C0, y_top, jnp.where(o < _C218, y_mid, y_bot))


def _sc_compact(d2pad, selm):
    mesh = plsc.VectorSubcoreMesh(core_axis_name="c", subcore_axis_name="s")

    @functools.partial(
        pl.kernel,
        mesh=mesh,
        out_type=jax.ShapeDtypeStruct((_NNZ,), jnp.float32),
        compiler_params=pltpu.CompilerParams(needs_layout_passes=False),
        scratch_types=[
            pltpu.VMEM((_RING,), jnp.float32),
            pltpu.VMEM((_SUBBLK,), jnp.int32),
            pltpu.VMEM((_SUBBLK,), jnp.int32),
            pltpu.VMEM((_SUBBLK,), jnp.float32),
            pltpu.VMEM((_SUBBLK,), jnp.float32),
            pltpu.SemaphoreType.DMA,
            pltpu.SemaphoreType.DMA,
        ],
    )
    def gather_kernel(d2_hbm, selm_hbm, out_hbm, plane_v,
                      ia, ib, da, db, isem, osem):
        wid = lax.axis_index("s") * 2 + lax.axis_index("c")
        obase = wid * (_ROWS_PER_W * _LANES)

        def o_of(i):
            return jnp.minimum(obase + i * _SUBBLK, _OCLAMP)

        y0 = _yfun(obase)
        pltpu.sync_copy(d2_hbm.at[pl.ds(y0 * _PLANE, _PLANE)],
                        plane_v.at[pl.ds((y0 % 2) * _PLANE, _PLANE)])
        pltpu.async_copy(selm_hbm.at[pl.ds(o_of(0), _SUBBLK)], ia, isem)

        idx_bufs = (ia, ib)
        dat_bufs = (da, db)

        def pair(p, ycur):
            stores = []
            for q in range(2):
                i = 2 * p + q
                idx_v = idx_bufs[q]
                dat_v = dat_bufs[q]
                pltpu.make_async_copy(
                    selm_hbm.at[pl.ds(o_of(i), _SUBBLK)], idx_v, isem).wait()
                pltpu.async_copy(
                    selm_hbm.at[pl.ds(o_of(i + 1), _SUBBLK)],
                    idx_bufs[1 - q], isem)
                o = o_of(i)
                ynew = _yfun(o + _SUBBLK - 1)

                @pl.when(ynew != ycur)
                def _():
                    pltpu.sync_copy(
                        d2_hbm.at[pl.ds(ynew * _PLANE, _PLANE)],
                        plane_v.at[pl.ds((ynew % 2) * _PLANE, _PLANE)])

                def gv(v, c):
                    for u in range(8):
                        off = v * 128 + u * 16
                        idx = idx_v[pl.ds(off, 16)]
                        dat_v[pl.ds(off, 16)] = plsc.load_gather(
                            plane_v, [idx])
                    return c

                lax.fori_loop(0, _SUBBLK // 128, gv, 0)
                stores.append(pltpu.async_copy(
                    dat_v, out_hbm.at[pl.ds(o, _SUBBLK)], osem))
                ycur = ynew
            for st in stores:
                st.wait()
            return ycur

        lax.fori_loop(0, _NSB // 2, pair, y0)
        pltpu.make_async_copy(
            selm_hbm.at[pl.ds(o_of(_NSB), _SUBBLK)], ia, isem).wait()

    return gather_kernel(d2pad, selm)


def kernel(image):
    d2pad = _dense_weights(image).reshape(-1)
    vals = _sc_compact(d2pad, jnp.asarray(_SELM))
    return jnp.asarray(_INDICES), vals


# speedup vs baseline: 7.0962x; 1.1173x over previous
import functools

import numpy as np
import jax
import jax.numpy as jnp
from jax import lax
from jax.experimental import pallas as pl
from jax.experimental.pallas import tpu as pltpu
from jax.experimental.pallas import tpu_sc as plsc

H = W = 224
N = H * W
_RADII = [1, 2, 3, 4, 5, 6]
_RW = [1.0, 0.6, 0.4, 0.3, 0.2, 0.1]
_TAU = 0.15


def _build_static():
    d = {}
    for r, w in zip(_RADII, _RW):
        for dy in range(-r, r + 1):
            for dx in range(-r, r + 1):
                if (dx == 0 and dy == 0) or dx * dx + dy * dy > r * r:
                    continue
                d[(dy, dx)] = d.get((dy, dx), 0.0) + w
    offs = sorted(d.items(), key=lambda kv: kv[0][0] * W + kv[0][1])
    slots = []
    diag_j = None
    for (dy, dx), w in offs:
        if dy * W + dx > 0 and diag_j is None:
            diag_j = len(slots)
            slots.append((0, 0, None))
        slots.append((dy, dx, w))
    if diag_j is None:
        diag_j = len(slots)
        slots.append((0, 0, None))
    k = len(slots)

    dys = np.array([s[0] for s in slots], np.int64)
    dxs = np.array([s[1] for s in slots], np.int64)
    deltas = dys * W + dxs

    yy, xx = np.meshgrid(np.arange(H, dtype=np.int64),
                         np.arange(W, dtype=np.int64), indexing="ij")
    yf = yy.reshape(-1)
    xf = xx.reshape(-1)
    ny = yf[:, None] + dys[None, :]
    nx = xf[:, None] + dxs[None, :]
    valid = (ny >= 0) & (ny < H) & (nx >= 0) & (nx < W)

    pix = np.arange(N, dtype=np.int64)
    rows = np.broadcast_to(pix[:, None], (N, k))[valid]
    cols = (pix[:, None] + deltas[None, :])[valid]
    jj = np.broadcast_to(np.arange(k, dtype=np.int64)[None, :], (N, k))
    d2idx = (yf[:, None] * k + jj) * 256 + xf[:, None]
    sel = d2idx[valid]

    indices = np.stack([rows, cols]).astype(np.int32)
    rowlens = valid.reshape(H, W * k).sum(axis=1).astype(np.int64)
    return slots, diag_j, k, indices, sel.astype(np.int32), rowlens


_SLOTS, _DIAG_J, _K, _INDICES, _SEL, _ROWLENS = _build_static()
_NNZ = _SEL.shape[0]

_BY = 32
_WPAD = 256
_WOUT = 256

_BY_GROUPS = {}
for _j, (_dy, _dx, _w) in enumerate(_SLOTS):
    if _w is not None:
        _BY_GROUPS.setdefault(_dy, []).append((_j, _dx, _w))


def _stencil_body(planes_ref, out_ref):
    yb = pl.program_id(0) * _BY
    tall = [planes_ref[ch, pl.ds(yb, _BY + 16), :] for ch in range(3)]
    ctr = [t[6:6 + _BY, 6:6 + W] for t in tall]
    rowi = jax.lax.broadcasted_iota(jnp.int32, (_BY, W), 0) + yb
    coli = jax.lax.broadcasted_iota(jnp.int32, (_BY, W), 1)
    one = jnp.ones((_BY, W), jnp.float32)
    zero = jnp.zeros((_BY, W), jnp.float32)
    oky = {dy: (jnp.where((rowi + dy >= 0) & (rowi + dy < H), one, zero)
                if dy else one)
           for dy in range(-6, 7)}
    okx = {dx: (jnp.where((coli + dx >= 0) & (coli + dx < W), one, zero)
                if dx else one)
           for dx in range(-6, 7)}
    accs = [zero, zero, zero, zero]
    for dy, group in sorted(_BY_GROUPS.items()):
        rsh = [t[6 + dy:6 + dy + _BY, :] for t in tall]
        for j, dx, w in group:
            sh = [r[:, 6 + dx:6 + dx + W] for r in rsh]
            d2 = ((ctr[0] - sh[0]) ** 2 + (ctr[1] - sh[1]) ** 2
                  + (ctr[2] - sh[2]) ** 2)
            ew = w * jnp.exp(jnp.sqrt(d2) * (-1.0 / _TAU))
            ewm = ew * (oky[dy] * okx[dx])
            accs[j % 4] = accs[j % 4] + ewm
            out_ref[:, j * _WOUT:j * _WOUT + W] = -ewm
    out_ref[:, _DIAG_J * _WOUT:_DIAG_J * _WOUT + W] = (
        (accs[0] + accs[1]) + (accs[2] + accs[3]))


def _dense_weights(image, interpret=False):
    img = image.astype(jnp.float32)
    planes = jnp.transpose(img, (2, 0, 1))
    planes = jnp.pad(planes, ((0, 0), (6, _BY + 10), (6, _WPAD - W - 6)))
    return pl.pallas_call(
        _stencil_body,
        grid=(H // _BY + 1,),
        in_specs=[pl.BlockSpec((3, H + _BY + 16, _WPAD), lambda i: (0, 0, 0))],
        out_specs=pl.BlockSpec((_BY, _K * _WOUT), lambda i: (i, 0)),
        out_shape=jax.ShapeDtypeStruct((H + _BY, _K * _WOUT), jnp.float32),
        interpret=interpret,
    )(planes)


_LANES = 128
_NW = 32
_PLANE = _K * _WOUT
_RING = 2 * _PLANE
_SB = 80
_SUBBLK = _SB * _LANES
_NSB = 18
_ROWS_PER_W = 1360
_OCLAMP = _NNZ - _SUBBLK

_SELM = (_SEL % _RING).astype(np.int32)

_T = np.concatenate([[0], np.cumsum(_ROWLENS)]).astype(np.int64)
_C0 = int(_T[6])
_LMID = int(_T[7] - _T[6])
_C218 = int(_T[218])
assert all(int(_T[y + 1] - _T[y]) == _LMID for y in range(6, 218))
assert int(_ROWLENS.min()) > _SUBBLK
_TTOP = [int(_T[y]) for y in range(1, 6)]
_TBOT = [int(_T[y]) for y in range(219, 225)]


def _yfun(o):
    y_top = sum((o >= t).astype(jnp.int32) for t in _TTOP)
    y_mid = (6 + (o - _C0) // _LMID).astype(jnp.int32)
    y_bot = 218 + sum((o >= t).astype(jnp.int32) for t in _TBOT)
    return jnp.where(o < _C0, y_top, jnp.where(o < _C218, y_mid, y_bot))


def _sc_compact(d2pad, selm):
    mesh = plsc.VectorSubcoreMesh(core_axis_name="c", subcore_axis_name="s")

    @functools.partial(
        pl.kernel,
        mesh=mesh,
        out_type=jax.ShapeDtypeStruct((_NNZ,), jnp.float32),
        compiler_params=pltpu.CompilerParams(needs_layout_passes=False),
        scratch_types=[
            pltpu.VMEM((_RING,), jnp.float32),
            pltpu.VMEM((_SUBBLK,), jnp.int32),
            pltpu.VMEM((_SUBBLK,), jnp.int32),
            pltpu.VMEM((_SUBBLK,), jnp.float32),
            pltpu.VMEM((_SUBBLK,), jnp.float32),
            pltpu.SemaphoreType.DMA,
            pltpu.SemaphoreType.DMA,
        ],
    )
    def gather_kernel(d2_hbm, selm_hbm, out_hbm, plane_v,
                      ia, ib, da, db, isem, osem):
        wid = lax.axis_index("s") * 2 + lax.axis_index("c")
        obase = wid * (_ROWS_PER_W * _LANES)

        def o_of(i):
            return jnp.minimum(obase + i * _SUBBLK, _OCLAMP)

        y0 = _yfun(obase)
        pltpu.sync_copy(d2_hbm.at[pl.ds(y0 * _PLANE, _PLANE)],
                        plane_v.at[pl.ds((y0 % 2) * _PLANE, _PLANE)])
        pltpu.async_copy(selm_hbm.at[pl.ds(o_of(0), _SUBBLK)], ia, isem)

        idx_bufs = (ia, ib)
        dat_bufs = (da, db)

        def pair(p, ycur):
            stores = []
            for q in range(2):
                i = 2 * p + q
                idx_v = idx_bufs[q]
                dat_v = dat_bufs[q]
                pltpu.make_async_copy(
                    selm_hbm.at[pl.ds(o_of(i), _SUBBLK)], idx_v, isem).wait()
                pltpu.async_copy(
                    selm_hbm.at[pl.ds(o_of(i + 1), _SUBBLK)],
                    idx_bufs[1 - q], isem)
                o = o_of(i)
                ynew = _yfun(o + _SUBBLK - 1)

                @pl.when(ynew != ycur)
                def _():
                    pltpu.sync_copy(
                        d2_hbm.at[pl.ds(ynew * _PLANE, _PLANE)],
                        plane_v.at[pl.ds((ynew % 2) * _PLANE, _PLANE)])

                @plsc.parallel_loop(0, _SUBBLK, 16, unroll=8)
                def _gather(off):
                    idx = idx_v[pl.ds(off, 16)]
                    dat_v[pl.ds(off, 16)] = plsc.load_gather(plane_v, [idx])
                stores.append(pltpu.async_copy(
                    dat_v, out_hbm.at[pl.ds(o, _SUBBLK)], osem))
                ycur = ynew
            for st in stores:
                st.wait()
            return ycur

        lax.fori_loop(0, _NSB // 2, pair, y0)
        pltpu.make_async_copy(
            selm_hbm.at[pl.ds(o_of(_NSB), _SUBBLK)], ia, isem).wait()

    return gather_kernel(d2pad, selm)


def kernel(image):
    d2pad = _dense_weights(image).reshape(-1)
    vals = _sc_compact(d2pad, jnp.asarray(_SELM))
    return jnp.asarray(_INDICES), vals
